# Initial kernel scaffold; baseline (speedup 1.0000x reference)
import jax
import jax.numpy as jnp
from jax.experimental import pallas as pl

N = 10000
E = 320000
D_IN = 128
HID = 256
D_OUT = 128

BM = 500


def _mm_body(x_ref, w_ref, o_ref):
    o_ref[...] = jnp.dot(x_ref[...], w_ref[...], preferred_element_type=jnp.float32)


def _matmul(x, w):
    m, k = x.shape
    _, n = w.shape
    return pl.pallas_call(
        _mm_body,
        grid=(m // BM,),
        in_specs=[
            pl.BlockSpec((BM, k), lambda i: (i, 0)),
            pl.BlockSpec((k, n), lambda i: (0, 0)),
        ],
        out_specs=pl.BlockSpec((BM, n), lambda i: (i, 0)),
        out_shape=jax.ShapeDtypeStruct((m, n), jnp.float32),
    )(x, w)


def _spmm(edge_index, edge_weight, h, n_nodes):
    src = edge_index[0]
    dst = edge_index[1]
    msgs = jnp.take(h, src, axis=0) * edge_weight[:, None]
    return jax.ops.segment_sum(msgs, dst, num_segments=n_nodes)


def kernel(x, edge_index, edge_weight, W1, b1, W2, b2):
    h = _matmul(x, W1)
    h = _spmm(edge_index, edge_weight, h, N)
    h = jax.nn.relu(h + b1)
    h2 = _matmul(h, W2)
    h2 = _spmm(edge_index, edge_weight, h2, N)
    return jax.nn.relu(h2 + b2)


# speedup vs baseline: 1.0730x; 1.0730x over previous
import jax
import jax.numpy as jnp
from jax.experimental import pallas as pl

N = 10000
E = 320000
D_IN = 128
HID = 256
D_OUT = 128

BM = 400


def _mm_body(x_ref, w_ref, o_ref):
    o_ref[...] = jnp.dot(x_ref[...], w_ref[...], preferred_element_type=jnp.float32)


def _matmul(x, w):
    m, k = x.shape
    _, n = w.shape
    return pl.pallas_call(
        _mm_body,
        grid=(m // BM,),
        in_specs=[
            pl.BlockSpec((BM, k), lambda i: (i, 0)),
            pl.BlockSpec((k, n), lambda i: (0, 0)),
        ],
        out_specs=pl.BlockSpec((BM, n), lambda i: (i, 0)),
        out_shape=jax.ShapeDtypeStruct((m, n), jnp.float32),
    )(x, w)


def _spmm(edge_index, edge_weight, h, n_nodes):
    src = edge_index[0]
    dst = edge_index[1]
    msgs = jnp.take(h, src, axis=0) * edge_weight[:, None]
    return jax.ops.segment_sum(msgs, dst, num_segments=n_nodes)


def kernel(x, edge_index, edge_weight, W1, b1, W2, b2):
    h = _matmul(x, W1)
    h = _spmm(edge_index, edge_weight, h, N)
    h = jax.nn.relu(h + b1)
    h2 = _matmul(h, W2)
    h2 = _spmm(edge_index, edge_weight, h2, N)
    return jax.nn.relu(h2 + b2)


# speedup vs baseline: 2.9422x; 2.7419x over previous
import dataclasses
import functools

import jax
import jax.numpy as jnp
from jax import lax
from jax.experimental import pallas as pl
from jax.experimental.pallas import tpu as pltpu
from jax.experimental.pallas import tpu_sc as plsc

N = 10000
E = 320000
D_IN = 128
HID = 256
D_OUT = 128

NC = 2
NS = 16
B = 80
BM = 400
RPS = 624
RPS_LAST = N - (NS - 1) * RPS

_mesh = plsc.VectorSubcoreMesh(core_axis_name="c", subcore_axis_name="s")

_sc_params = pltpu.CompilerParams()
if "needs_layout_passes" in pltpu.CompilerParams.__dataclass_fields__:
    _sc_params = dataclasses.replace(_sc_params, needs_layout_passes=False)


def _mm1_body(x_ref, w_ref, o_ref):
    o_ref[...] = jnp.dot(x_ref[...], w_ref[...], preferred_element_type=jnp.float32)


def _mm1(x, W1):
    nb = N // BM
    return pl.pallas_call(
        _mm1_body,
        grid=(NC, nb),
        in_specs=[
            pl.BlockSpec((BM, D_IN), lambda j, i: (i, 0)),
            pl.BlockSpec((D_IN, HID // NC), lambda j, i: (0, j)),
        ],
        out_specs=pl.BlockSpec((BM, HID // NC), lambda j, i: (j * (N // BM) + i, 0)),
        out_shape=jax.ShapeDtypeStruct((NC * N, HID // NC), jnp.float32),
    )(x, W1)


def _mm2_body(h0_ref, h1_ref, b1_ref, w2_ref, o_ref):
    a0 = jnp.maximum(h0_ref[...] + b1_ref[0:1, 0:128], 0.0)
    a1 = jnp.maximum(h1_ref[...] + b1_ref[0:1, 128:256], 0.0)
    a = jnp.concatenate([a0, a1], axis=1)
    o_ref[...] = jnp.dot(a, w2_ref[...], preferred_element_type=jnp.float32)


def _mm2(o1, b1r, W2):
    nb = N // BM
    return pl.pallas_call(
        _mm2_body,
        grid=(nb,),
        in_specs=[
            pl.BlockSpec((BM, HID // NC), lambda i: (i, 0)),
            pl.BlockSpec((BM, HID // NC), lambda i: (N // BM + i, 0)),
            pl.BlockSpec((1, HID), lambda i: (0, 0)),
            pl.BlockSpec((HID, D_OUT), lambda i: (0, 0)),
        ],
        out_specs=pl.BlockSpec((BM, D_OUT), lambda i: (i, 0)),
        out_shape=jax.ShapeDtypeStruct((N, D_OUT), jnp.float32),
    )(o1, o1, b1r, W2)


def _final_body(p0_ref, p1_ref, b2_ref, o_ref):
    o_ref[...] = jnp.maximum(p0_ref[...] + p1_ref[...] + b2_ref[0:1, :], 0.0)


def _final(o2, b2r):
    nb = N // BM
    return pl.pallas_call(
        _final_body,
        grid=(nb,),
        in_specs=[
            pl.BlockSpec((BM, D_OUT), lambda i: (i, 0)),
            pl.BlockSpec((BM, D_OUT), lambda i: (N // BM + i, 0)),
            pl.BlockSpec((1, D_OUT), lambda i: (0, 0)),
        ],
        out_specs=pl.BlockSpec((BM, D_OUT), lambda i: (i, 0)),
        out_shape=jax.ShapeDtypeStruct((N, D_OUT), jnp.float32),
    )(o2, o2, b2r)


def _make_spmm(H, mode):
    if mode == "feat":
        epw = E // NS
    else:
        epw = E // (NC * NS)
    nblk = epw // B
    assert epw % B == 0

    @functools.partial(
        pl.kernel,
        out_type=jax.ShapeDtypeStruct((NC * N, H), jnp.float32),
        mesh=_mesh,
        scratch_types=[
            pltpu.VMEM((B,), jnp.int32),
            pltpu.VMEM((1, B), jnp.int32),
            pltpu.VMEM((1, B), jnp.float32),
            pltpu.VMEM((B, H), jnp.float32),
            pltpu.VMEM_SHARED((N, H), jnp.float32),
            pltpu.SemaphoreType.DMA,
        ],
        compiler_params=_sc_params,
    )
    def spmm(table, src_hbm, dst_hbm, ew_hbm, z_hbm, out,
             src_v, dst_v, ew_v, rows_v, acc, sem):
        c = lax.axis_index("c")
        s = lax.axis_index("s")

        @pl.when(s < NS - 1)
        def _():
            pltpu.sync_copy(z_hbm.at[pl.ds(0, RPS)], acc.at[pl.ds(s * RPS, RPS)])

        @pl.when(s == NS - 1)
        def _():
            pltpu.sync_copy(z_hbm, acc.at[pl.ds((NS - 1) * RPS, RPS_LAST)])

        plsc.subcore_barrier()

        if mode == "feat":
            chunk = s * epw
        else:
            chunk = (s * NC + c) * epw

        @pl.loop(0, nblk)
        def _blk(blk):
            base = chunk + blk * B
            pltpu.sync_copy(src_hbm.at[pl.ds(base, B)], src_v)
            pltpu.sync_copy(dst_hbm.at[pl.ds(base, B)], dst_v.at[0])
            pltpu.sync_copy(ew_hbm.at[pl.ds(base, B)], ew_v.at[0])
            if mode == "feat":
                off = c * N
                for k in range(B // 16):
                    sl = pl.ds(k * 16, 16)
                    src_v[sl] = src_v[sl] + off
            pltpu.async_copy(table.at[src_v], rows_v, sem).wait()

            @pl.loop(0, B)
            def _edge(e):
                w = plsc.load_gather(
                    ew_v,
                    [jnp.zeros((16,), jnp.int32),
                     jnp.full((16,), e, dtype=jnp.int32)],
                )
                for f in range(H // 16):
                    fs = pl.ds(f * 16, 16)
                    rows_v[e, fs] = rows_v[e, fs] * w

            pltpu.sync_copy(rows_v, acc.at[dst_v.at[0]], add=True)

        plsc.subcore_barrier()

        @pl.when(s < NS - 1)
        def _():
            pltpu.sync_copy(acc.at[pl.ds(s * RPS, RPS)],
                            out.at[pl.ds(c * N + s * RPS, RPS)])

        @pl.when(s == NS - 1)
        def _():
            pltpu.sync_copy(acc.at[pl.ds((NS - 1) * RPS, RPS_LAST)],
                            out.at[pl.ds(c * N + (NS - 1) * RPS, RPS_LAST)])

    return spmm


_spmm_feat = _make_spmm(HID // NC, "feat")
_spmm_edge = _make_spmm(D_OUT, "edge")


def kernel(x, edge_index, edge_weight, W1, b1, W2, b2):
    src = edge_index[0]
    dst = edge_index[1]
    z1 = jnp.zeros((RPS_LAST, HID // NC), jnp.float32)
    z2 = jnp.zeros((RPS_LAST, D_OUT), jnp.float32)

    h1 = _mm1(x, W1)
    o1 = _spmm_feat(h1, src, dst, edge_weight, z1)
    h2 = _mm2(o1, b1.reshape(1, HID), W2)
    o2 = _spmm_edge(h2, src, dst, edge_weight, z2)
    return _final(o2, b2.reshape(1, D_OUT))


# speedup vs baseline: 7.4221x; 2.5227x over previous
import dataclasses
import functools

import jax
import jax.numpy as jnp
from jax import lax
from jax.experimental import pallas as pl
from jax.experimental.pallas import tpu as pltpu
from jax.experimental.pallas import tpu_sc as plsc

N = 10000
E = 320000
D_IN = 128
HID = 256
D_OUT = 128

NC = 2
NS = 16
BM = 400
RPS = 624
RPS_LAST = N - (NS - 1) * RPS

_mesh = plsc.VectorSubcoreMesh(core_axis_name="c", subcore_axis_name="s")

_sc_params = pltpu.CompilerParams()
if "needs_layout_passes" in pltpu.CompilerParams.__dataclass_fields__:
    _sc_params = dataclasses.replace(_sc_params, needs_layout_passes=False)


def _mm1_body(x_ref, w_ref, o_ref):
    o_ref[...] = jnp.dot(x_ref[...], w_ref[...], preferred_element_type=jnp.float32)


def _mm1(x, W1):
    nb = N // BM
    return pl.pallas_call(
        _mm1_body,
        grid=(NC, nb),
        in_specs=[
            pl.BlockSpec((BM, D_IN), lambda j, i: (i, 0)),
            pl.BlockSpec((D_IN, HID // NC), lambda j, i: (0, j)),
        ],
        out_specs=pl.BlockSpec((BM, HID // NC), lambda j, i: (j * (N // BM) + i, 0)),
        out_shape=jax.ShapeDtypeStruct((NC * N, HID // NC), jnp.float32),
    )(x, W1)


def _mm2_body(h0_ref, h1_ref, b1_ref, w2_ref, o_ref):
    a0 = jnp.maximum(h0_ref[...] + b1_ref[0:1, 0:128], 0.0)
    a1 = jnp.maximum(h1_ref[...] + b1_ref[0:1, 128:256], 0.0)
    a = jnp.concatenate([a0, a1], axis=1)
    o_ref[...] = jnp.dot(a, w2_ref[...], preferred_element_type=jnp.float32)


def _mm2(o1, b1r, W2):
    nb = N // BM
    return pl.pallas_call(
        _mm2_body,
        grid=(nb,),
        in_specs=[
            pl.BlockSpec((BM, HID // NC), lambda i: (i, 0)),
            pl.BlockSpec((BM, HID // NC), lambda i: (N // BM + i, 0)),
            pl.BlockSpec((1, HID), lambda i: (0, 0)),
            pl.BlockSpec((HID, D_OUT), lambda i: (0, 0)),
        ],
        out_specs=pl.BlockSpec((BM, D_OUT), lambda i: (i, 0)),
        out_shape=jax.ShapeDtypeStruct((N, D_OUT), jnp.float32),
    )(o1, o1, b1r, W2)


def _final_body(p0_ref, p1_ref, b2_ref, o_ref):
    o_ref[...] = jnp.maximum(p0_ref[...] + p1_ref[...] + b2_ref[0:1, :], 0.0)


def _final(o2, b2r):
    nb = N // BM
    return pl.pallas_call(
        _final_body,
        grid=(nb,),
        in_specs=[
            pl.BlockSpec((BM, D_OUT), lambda i: (i, 0)),
            pl.BlockSpec((BM, D_OUT), lambda i: (N // BM + i, 0)),
            pl.BlockSpec((1, D_OUT), lambda i: (0, 0)),
        ],
        out_specs=pl.BlockSpec((BM, D_OUT), lambda i: (i, 0)),
        out_shape=jax.ShapeDtypeStruct((N, D_OUT), jnp.float32),
    )(o2, o2, b2r)


def _make_spmm(H, B, mode):
    if mode == "feat":
        epw = E // NS
    else:
        epw = E // (NC * NS)
    nblk = epw // B
    assert epw % B == 0 and nblk % 2 == 0 and B % 8 == 0 and B <= 128
    half = nblk // 2

    @functools.partial(
        pl.kernel,
        out_type=jax.ShapeDtypeStruct((NC * N, H), jnp.float32),
        mesh=_mesh,
        scratch_types=[
            pltpu.VMEM((2, B), jnp.int32),
            pltpu.VMEM((2, B), jnp.int32),
            pltpu.VMEM((2, B), jnp.float32),
            pltpu.VMEM((2, B, H), jnp.float32),
            pltpu.VMEM_SHARED((N, H), jnp.float32),
            pltpu.SemaphoreType.DMA,
            pltpu.SemaphoreType.DMA,
            pltpu.SemaphoreType.DMA,
            pltpu.SemaphoreType.DMA,
            pltpu.SemaphoreType.DMA,
            pltpu.SemaphoreType.DMA,
            pltpu.SemaphoreType.DMA,
            pltpu.SemaphoreType.DMA,
        ],
        compiler_params=_sc_params,
    )
    def spmm(table, src_hbm, dst_hbm, ew_hbm, z_hbm, out,
             src_v, dst_v, ew_v, rows_v, acc,
             gs0, gs1, is0, is1, ds0, ds1, ss0, ss1):
        c = lax.axis_index("c")
        s = lax.axis_index("s")
        gsem = (gs0, gs1)
        isem = (is0, is1)
        dsem = (ds0, ds1)
        scsem = (ss0, ss1)

        @pl.when(s < NS - 1)
        def _():
            pltpu.sync_copy(z_hbm.at[pl.ds(0, RPS)], acc.at[pl.ds(s * RPS, RPS)])

        @pl.when(s == NS - 1)
        def _():
            pltpu.sync_copy(z_hbm, acc.at[pl.ds((NS - 1) * RPS, RPS_LAST)])

        plsc.subcore_barrier()

        if mode == "feat":
            chunk = s * epw
            off = c * N
        else:
            chunk = (s * NC + c) * epw

        def start_idx(blk, p):
            b = chunk + blk * B
            pltpu.async_copy(src_hbm.at[pl.ds(b, B)], src_v.at[p], isem[p])
            pltpu.async_copy(ew_hbm.at[pl.ds(b, B)], ew_v.at[p], isem[p])

        def wait_idx(blk, p):
            b = chunk + blk * B
            pltpu.make_async_copy(src_hbm.at[pl.ds(b, B)], src_v.at[p], isem[p]).wait()
            pltpu.make_async_copy(ew_hbm.at[pl.ds(b, B)], ew_v.at[p], isem[p]).wait()

        def adjust(p):
            if mode == "feat":
                for k in range(B // 16):
                    sl = (p, pl.ds(k * 16, 16))
                    src_v[sl] = src_v[sl] + off

        def start_dst(blk, p):
            b = chunk + blk * B
            pltpu.async_copy(dst_hbm.at[pl.ds(b, B)], dst_v.at[p], dsem[p])

        def wait_dst(blk, p):
            b = chunk + blk * B
            pltpu.make_async_copy(dst_hbm.at[pl.ds(b, B)], dst_v.at[p], dsem[p]).wait()

        def start_gather(p):
            pltpu.async_copy(table.at[src_v.at[p]], rows_v.at[p], gsem[p])

        def wait_gather(p):
            pltpu.make_async_copy(table.at[src_v.at[p]], rows_v.at[p], gsem[p]).wait()

        def start_scatter(p):
            pltpu.async_copy(rows_v.at[p], acc.at[dst_v.at[p]], scsem[p], add=True)

        def wait_scatter(p):
            pltpu.make_async_copy(rows_v.at[p], acc.at[dst_v.at[p]], scsem[p]).wait()

        def multiply(p):
            @plsc.parallel_loop(0, B, unroll=2)
            def _edge(e):
                w = plsc.load_gather(
                    ew_v,
                    [jnp.full((16,), p, dtype=jnp.int32),
                     jnp.full((16,), e, dtype=jnp.int32)],
                )
                for f in range(H // 16):
                    fs = (p, e, pl.ds(f * 16, 16))
                    rows_v[fs] = rows_v[fs] * w

        start_idx(0, 0)
        wait_idx(0, 0)
        adjust(0)
        start_gather(0)
        start_dst(0, 0)
        start_idx(1, 1)

        @pl.loop(0, half)
        def _t(t):
            not_last = t < half - 1

            blk = 2 * t
            wait_idx(blk + 1, 1)
            adjust(1)

            @pl.when(t > 0)
            def _():
                wait_scatter(1)

            start_gather(1)
            start_dst(blk + 1, 1)
            wait_gather(0)
            multiply(0)

            @pl.when(not_last)
            def _():
                start_idx(blk + 2, 0)

            wait_dst(blk, 0)
            start_scatter(0)

            blk1 = 2 * t + 1

            @pl.when(not_last)
            def _():
                wait_idx(blk1 + 1, 0)
                adjust(0)

            wait_scatter(0)

            @pl.when(not_last)
            def _():
                start_gather(0)
                start_dst(blk1 + 1, 0)

            wait_gather(1)
            multiply(1)

            @pl.when(not_last)
            def _():
                start_idx(blk1 + 2, 1)

            wait_dst(blk1, 1)
            start_scatter(1)

        wait_scatter(1)
        plsc.subcore_barrier()

        @pl.when(s < NS - 1)
        def _():
            pltpu.sync_copy(acc.at[pl.ds(s * RPS, RPS)],
                            out.at[pl.ds(c * N + s * RPS, RPS)])

        @pl.when(s == NS - 1)
        def _():
            pltpu.sync_copy(acc.at[pl.ds((NS - 1) * RPS, RPS_LAST)],
                            out.at[pl.ds(c * N + (NS - 1) * RPS, RPS_LAST)])

    return spmm


_spmm_feat = _make_spmm(HID // NC, 80, "feat")
_spmm_edge = _make_spmm(D_OUT, 40, "edge")


def kernel(x, edge_index, edge_weight, W1, b1, W2, b2):
    src = edge_index[0]
    dst = edge_index[1]
    z1 = jnp.zeros((RPS_LAST, HID // NC), jnp.float32)
    z2 = jnp.zeros((RPS_LAST, D_OUT), jnp.float32)

    h1 = _mm1(x, W1)
    o1 = _spmm_feat(h1, src, dst, edge_weight, z1)
    h2 = _mm2(o1, b1.reshape(1, HID), W2)
    o2 = _spmm_edge(h2, src, dst, edge_weight, z2)
    return _final(o2, b2.reshape(1, D_OUT))


# speedup vs baseline: 8.4050x; 1.1324x over previous
import dataclasses
import functools

import jax
import jax.numpy as jnp
from jax import lax
from jax.experimental import pallas as pl
from jax.experimental.pallas import tpu as pltpu
from jax.experimental.pallas import tpu_sc as plsc

N = 10000
E = 320000
D_IN = 128
HID = 256
D_OUT = 128

NC = 2
NS = 16
BM = 400
RPS = 624
RPS_LAST = N - (NS - 1) * RPS

_mesh = plsc.VectorSubcoreMesh(core_axis_name="c", subcore_axis_name="s")

_sc_params = pltpu.CompilerParams()
if "needs_layout_passes" in pltpu.CompilerParams.__dataclass_fields__:
    _sc_params = dataclasses.replace(_sc_params, needs_layout_passes=False)


def _mm1_body(x_ref, w_ref, o_ref):
    o_ref[...] = jnp.dot(x_ref[...], w_ref[...], preferred_element_type=jnp.float32)


def _mm1(x, W1):
    nb = N // BM
    return pl.pallas_call(
        _mm1_body,
        grid=(NC, nb),
        in_specs=[
            pl.BlockSpec((BM, D_IN), lambda j, i: (i, 0)),
            pl.BlockSpec((D_IN, HID // NC), lambda j, i: (0, j)),
        ],
        out_specs=pl.BlockSpec((BM, HID // NC), lambda j, i: (j * (N // BM) + i, 0)),
        out_shape=jax.ShapeDtypeStruct((NC * N, HID // NC), jnp.float32),
    )(x, W1)


def _mm2_body(h0_ref, h1_ref, b1_ref, w2_ref, o_ref):
    a0 = jnp.maximum(h0_ref[...] + b1_ref[0:1, 0:128], 0.0)
    a1 = jnp.maximum(h1_ref[...] + b1_ref[0:1, 128:256], 0.0)
    a = jnp.concatenate([a0, a1], axis=1)
    o_ref[...] = jnp.dot(a, w2_ref[...], preferred_element_type=jnp.float32)


def _mm2(o1, b1r, W2):
    nb = N // BM
    return pl.pallas_call(
        _mm2_body,
        grid=(nb,),
        in_specs=[
            pl.BlockSpec((BM, HID // NC), lambda i: (i, 0)),
            pl.BlockSpec((BM, HID // NC), lambda i: (N // BM + i, 0)),
            pl.BlockSpec((1, HID), lambda i: (0, 0)),
            pl.BlockSpec((HID, D_OUT), lambda i: (0, 0)),
        ],
        out_specs=pl.BlockSpec((BM, D_OUT), lambda i: (i, 0)),
        out_shape=jax.ShapeDtypeStruct((N, D_OUT), jnp.float32),
    )(o1, o1, b1r, W2)


def _final_body(p0_ref, p1_ref, b2_ref, o_ref):
    o_ref[...] = jnp.maximum(p0_ref[...] + p1_ref[...] + b2_ref[0:1, :], 0.0)


def _final(o2, b2r):
    nb = N // BM
    return pl.pallas_call(
        _final_body,
        grid=(nb,),
        in_specs=[
            pl.BlockSpec((BM, D_OUT), lambda i: (i, 0)),
            pl.BlockSpec((BM, D_OUT), lambda i: (N // BM + i, 0)),
            pl.BlockSpec((1, D_OUT), lambda i: (0, 0)),
        ],
        out_specs=pl.BlockSpec((BM, D_OUT), lambda i: (i, 0)),
        out_shape=jax.ShapeDtypeStruct((N, D_OUT), jnp.float32),
    )(o2, o2, b2r)


def _make_spmm(H, B, mode):
    if mode == "feat":
        epw = E // NS
    else:
        epw = E // (NC * NS)
    nblk = epw // B
    assert epw % B == 0 and B % 8 == 0 and B <= 128 and nblk >= 4
    pairs = nblk // 2
    odd = nblk % 2 == 1

    @functools.partial(
        pl.kernel,
        out_type=jax.ShapeDtypeStruct((NC * N, H), jnp.float32),
        mesh=_mesh,
        scratch_types=[
            pltpu.VMEM((2, B), jnp.int32),
            pltpu.VMEM((2, B), jnp.int32),
            pltpu.VMEM((2, B), jnp.float32),
            pltpu.VMEM((2, B, H), jnp.float32),
            pltpu.VMEM_SHARED((N, H), jnp.float32),
            pltpu.SemaphoreType.DMA,
            pltpu.SemaphoreType.DMA,
            pltpu.SemaphoreType.DMA,
            pltpu.SemaphoreType.DMA,
            pltpu.SemaphoreType.DMA,
            pltpu.SemaphoreType.DMA,
            pltpu.SemaphoreType.DMA,
            pltpu.SemaphoreType.DMA,
        ],
        compiler_params=_sc_params,
    )
    def spmm(table, src_hbm, dst_hbm, ew_hbm, z_hbm, out,
             src_v, dst_v, ew_v, rows_v, acc,
             gs0, gs1, is0, is1, ds0, ds1, ss0, ss1):
        c = lax.axis_index("c")
        s = lax.axis_index("s")
        gsem = (gs0, gs1)
        isem = (is0, is1)
        dsem = (ds0, ds1)
        scsem = (ss0, ss1)

        @pl.when(s < NS - 1)
        def _():
            pltpu.sync_copy(z_hbm.at[pl.ds(0, RPS)], acc.at[pl.ds(s * RPS, RPS)])

        @pl.when(s == NS - 1)
        def _():
            pltpu.sync_copy(z_hbm, acc.at[pl.ds((NS - 1) * RPS, RPS_LAST)])

        plsc.subcore_barrier()

        if mode == "feat":
            chunk = s * epw
            off = c * N
        else:
            chunk = (s * NC + c) * epw

        def start_idx(blk, p):
            b = chunk + blk * B
            pltpu.async_copy(src_hbm.at[pl.ds(b, B)], src_v.at[p], isem[p])
            pltpu.async_copy(ew_hbm.at[pl.ds(b, B)], ew_v.at[p], isem[p])

        def wait_idx(blk, p):
            b = chunk + blk * B
            pltpu.make_async_copy(src_hbm.at[pl.ds(b, B)], src_v.at[p], isem[p]).wait()
            pltpu.make_async_copy(ew_hbm.at[pl.ds(b, B)], ew_v.at[p], isem[p]).wait()

        def adjust(p):
            if mode == "feat":
                for k in range(B // 16):
                    sl = (p, pl.ds(k * 16, 16))
                    src_v[sl] = src_v[sl] + off

        def start_dst(blk, p):
            b = chunk + blk * B
            pltpu.async_copy(dst_hbm.at[pl.ds(b, B)], dst_v.at[p], dsem[p])

        def wait_dst(blk, p):
            b = chunk + blk * B
            pltpu.make_async_copy(dst_hbm.at[pl.ds(b, B)], dst_v.at[p], dsem[p]).wait()

        def start_gather(p):
            pltpu.async_copy(table.at[src_v.at[p]], rows_v.at[p], gsem[p])

        def wait_gather(p):
            pltpu.make_async_copy(table.at[src_v.at[p]], rows_v.at[p], gsem[p]).wait()

        def start_scatter(p):
            pltpu.async_copy(rows_v.at[p], acc.at[dst_v.at[p]], scsem[p], add=True)

        def wait_scatter(p):
            pltpu.make_async_copy(rows_v.at[p], acc.at[dst_v.at[p]], scsem[p]).wait()

        def multiply(p):
            @plsc.parallel_loop(0, B // 16, unroll=1)
            def _grp(g):
                w16 = ew_v[p, pl.ds(g * 16, 16)]
                for j in range(16):
                    wj = w16.at[jnp.full((16,), j, dtype=jnp.int32)].get(
                        mode="promise_in_bounds")
                    e = g * 16 + j
                    for f in range(H // 16):
                        fs = (p, e, pl.ds(f * 16, 16))
                        rows_v[fs] = rows_v[fs] * wj

        start_idx(0, 0)
        wait_idx(0, 0)
        adjust(0)
        start_gather(0)
        start_dst(0, 0)
        start_idx(1, 1)

        @pl.loop(0, pairs)
        def _t(t):
            not_last = t < pairs - 1

            blk = 2 * t
            wait_idx(blk + 1, 1)
            adjust(1)

            @pl.when(t > 0)
            def _():
                wait_scatter(1)

            start_gather(1)
            start_dst(blk + 1, 1)
            wait_gather(0)
            multiply(0)

            if odd:
                start_idx(blk + 2, 0)
            else:
                @pl.when(not_last)
                def _():
                    start_idx(blk + 2, 0)

            wait_dst(blk, 0)
            start_scatter(0)

            blk1 = 2 * t + 1

            def _prep_next():
                wait_idx(blk1 + 1, 0)
                adjust(0)

            if odd:
                _prep_next()
            else:
                pl.when(not_last)(_prep_next)

            wait_scatter(0)

            def _launch_next():
                start_gather(0)
                start_dst(blk1 + 1, 0)

            if odd:
                _launch_next()
            else:
                pl.when(not_last)(_launch_next)

            wait_gather(1)
            multiply(1)

            @pl.when(not_last)
            def _():
                start_idx(blk1 + 2, 1)

            wait_dst(blk1, 1)
            start_scatter(1)

        if odd:
            wait_gather(0)
            multiply(0)
            wait_dst(nblk - 1, 0)
            start_scatter(0)
            wait_scatter(1)
            wait_scatter(0)
        else:
            wait_scatter(1)
        plsc.subcore_barrier()

        @pl.when(s < NS - 1)
        def _():
            pltpu.sync_copy(acc.at[pl.ds(s * RPS, RPS)],
                            out.at[pl.ds(c * N + s * RPS, RPS)])

        @pl.when(s == NS - 1)
        def _():
            pltpu.sync_copy(acc.at[pl.ds((NS - 1) * RPS, RPS_LAST)],
                            out.at[pl.ds(c * N + (NS - 1) * RPS, RPS_LAST)])

    return spmm


_spmm1 = _make_spmm(HID // NC, 80, "feat")
_spmm2 = _make_spmm(D_OUT, 80, "edge")


def kernel(x, edge_index, edge_weight, W1, b1, W2, b2):
    src = edge_index[0]
    dst = edge_index[1]
    z1 = jnp.zeros((RPS_LAST, HID // NC), jnp.float32)
    z2 = jnp.zeros((RPS_LAST, D_OUT), jnp.float32)

    h1 = _mm1(x, W1)
    o1 = _spmm1(h1, src, dst, edge_weight, z1)
    h2 = _mm2(o1, b1.reshape(1, HID), W2)
    o2 = _spmm2(h2, src, dst, edge_weight, z2)
    return _final(o2, b2.reshape(1, D_OUT))
